# Initial kernel scaffold; baseline (speedup 1.0000x reference)
import jax
import jax.numpy as jnp
from jax.experimental import pallas as pl


def kernel(x, coefficients):
    raise NotImplementedError("write your pallas kernel here")


# speedup vs baseline: 398.4820x; 398.4820x over previous
import functools

import jax
import jax.numpy as jnp
from jax import lax
from jax.experimental import pallas as pl
from jax.experimental.pallas import tpu as pltpu
from jax.experimental.pallas import tpu_sc as plsc

NUM_ACT = 2048
NUM_KNOT = 64
X_MIN = -4.0
X_MAX = 4.0
STEP = (X_MAX - X_MIN) / (NUM_KNOT - 1)
INV_STEP = 1.0 / STEP

ROWS = 16384
NW = 32
COLS_PER_W = NUM_ACT // NW
CHUNK_R = 256
N_CHUNKS = ROWS // CHUNK_R
GROUPS_PER_ROW = COLS_PER_W // 16


def _spline_body(x_hbm, coef_hbm, out_hbm, tab_v, xbuf, obuf):
    cid = lax.axis_index("c")
    sid = lax.axis_index("s")
    wid = sid * 2 + cid
    cb = wid * COLS_PER_W

    pltpu.sync_copy(coef_hbm.at[pl.ds(cb, COLS_PER_W)], tab_v)

    def chunk_body(g, carry):
        r0 = g * CHUNK_R
        pltpu.sync_copy(
            x_hbm.at[pl.ds(r0, CHUNK_R), pl.ds(cb, COLS_PER_W)], xbuf
        )

        def row_body(r, c2):
            for grp in range(GROUPS_PER_ROW):
                xv = xbuf[r, pl.ds(grp * 16, 16)]
                t = (xv - X_MIN) * INV_STEP
                tcl = jnp.minimum(jnp.maximum(t, 0.0), float(NUM_KNOT - 2))
                ki = tcl.astype(jnp.int32)
                frac = t - ki.astype(jnp.float32)
                colv = lax.iota(jnp.int32, 16) + (grp * 16)
                c0 = plsc.load_gather(tab_v, [colv, ki])
                c1 = plsc.load_gather(tab_v, [colv, ki + 1])
                obuf[r, pl.ds(grp * 16, 16)] = c1 * frac + c0 * (1.0 - frac)
            return c2

        lax.fori_loop(0, CHUNK_R, row_body, 0, unroll=2)
        pltpu.sync_copy(
            obuf, out_hbm.at[pl.ds(r0, CHUNK_R), pl.ds(cb, COLS_PER_W)]
        )
        return carry

    lax.fori_loop(0, N_CHUNKS, chunk_body, 0)


@jax.jit
def kernel(x, coefficients):
    mesh = plsc.VectorSubcoreMesh(core_axis_name="c", subcore_axis_name="s")
    run = pl.kernel(
        _spline_body,
        out_type=jax.ShapeDtypeStruct((ROWS, NUM_ACT), jnp.float32),
        mesh=mesh,
        scratch_types=[
            pltpu.VMEM((NUM_KNOT, NUM_KNOT), jnp.float32),
            pltpu.VMEM((CHUNK_R, COLS_PER_W), jnp.float32),
            pltpu.VMEM((CHUNK_R, COLS_PER_W), jnp.float32),
        ],
        compiler_params=pltpu.CompilerParams(use_tc_tiling_on_sc=False, needs_layout_passes=False),
        name="linear_spline_sc",
    )
    return run(x, coefficients)


# speedup vs baseline: 432.7208x; 1.0859x over previous
import jax
import jax.numpy as jnp
from jax import lax
from jax.experimental import pallas as pl
from jax.experimental.pallas import tpu as pltpu
from jax.experimental.pallas import tpu_sc as plsc

NUM_ACT = 2048
NUM_KNOT = 64
X_MIN = -4.0
X_MAX = 4.0
STEP = (X_MAX - X_MIN) / (NUM_KNOT - 1)
INV_STEP = 1.0 / STEP

ROWS = 16384
NW = 32
COLS_PER_W = NUM_ACT // NW
TAB_W = COLS_PER_W * NUM_KNOT
CHUNK_R = 256
N_CHUNKS = ROWS // CHUNK_R
GROUPS_PER_ROW = COLS_PER_W // 16


def _spline_body(x_hbm, coef_hbm, out_hbm, tab_v, xbufs, obufs, sems_in, sems_out):
    cid = lax.axis_index("c")
    sid = lax.axis_index("s")
    wid = sid * 2 + cid
    cb = wid * COLS_PER_W

    pltpu.sync_copy(coef_hbm.at[pl.ds(wid * TAB_W, TAB_W)], tab_v)

    def in_copy(g, b):
        return pltpu.make_async_copy(
            x_hbm.at[pl.ds(g * CHUNK_R, CHUNK_R), pl.ds(cb, COLS_PER_W)],
            xbufs[b],
            sems_in[b],
        )

    def out_copy(g, b):
        return pltpu.make_async_copy(
            obufs[b],
            out_hbm.at[pl.ds(g * CHUNK_R, CHUNK_R), pl.ds(cb, COLS_PER_W)],
            sems_out[b],
        )

    def compute_chunk(b):
        xbuf, obuf = xbufs[b], obufs[b]

        def row_body(r, carry):
            for grp in range(GROUPS_PER_ROW):
                xv = xbuf[r, pl.ds(grp * 16, 16)]
                t = (xv - X_MIN) * INV_STEP
                tcl = jnp.minimum(jnp.maximum(t, 0.0), float(NUM_KNOT - 2))
                ki = tcl.astype(jnp.int32)
                frac = t - ki.astype(jnp.float32)
                idx = (lax.iota(jnp.int32, 16) + grp * 16) * NUM_KNOT + ki
                c0 = plsc.load_gather(tab_v, [idx])
                c1 = plsc.load_gather(tab_v, [idx + 1])
                obuf[r, pl.ds(grp * 16, 16)] = c1 * frac + c0 * (1.0 - frac)
            return carry

        lax.fori_loop(0, CHUNK_R, row_body, 0, unroll=2)

    in_copy(0, 0).start()

    def pair_body(h, carry):
        for b in range(2):
            g = 2 * h + b
            @pl.when(g + 1 < N_CHUNKS)
            def _():
                in_copy(g + 1, 1 - b).start()

            in_copy(g, b).wait()

            @pl.when(g >= 2)
            def _():
                out_copy(jnp.maximum(g - 2, 0), b).wait()

            compute_chunk(b)
            out_copy(g, b).start()
        return carry

    lax.fori_loop(0, N_CHUNKS // 2, pair_body, 0)

    out_copy(N_CHUNKS - 2, 0).wait()
    out_copy(N_CHUNKS - 1, 1).wait()


@jax.jit
def kernel(x, coefficients):
    mesh = plsc.VectorSubcoreMesh(core_axis_name="c", subcore_axis_name="s")
    run = pl.kernel(
        _spline_body,
        out_type=jax.ShapeDtypeStruct((ROWS, NUM_ACT), jnp.float32),
        mesh=mesh,
        scratch_types=[
            pltpu.VMEM((TAB_W,), jnp.float32),
            [pltpu.VMEM((CHUNK_R, COLS_PER_W), jnp.float32) for _ in range(2)],
            [pltpu.VMEM((CHUNK_R, COLS_PER_W), jnp.float32) for _ in range(2)],
            [pltpu.SemaphoreType.DMA for _ in range(2)],
            [pltpu.SemaphoreType.DMA for _ in range(2)],
        ],
        compiler_params=pltpu.CompilerParams(
            use_tc_tiling_on_sc=False, needs_layout_passes=False
        ),
        name="linear_spline_sc",
    )
    return run(x, coefficients.reshape(-1))


# speedup vs baseline: 1379.3710x; 3.1877x over previous
import jax
import jax.numpy as jnp
from jax import lax
from jax.experimental import pallas as pl
from jax.experimental.pallas import tpu as pltpu
from jax.experimental.pallas import tpu_sc as plsc

NUM_ACT = 2048
NUM_KNOT = 64
X_MIN = -4.0
X_MAX = 4.0
STEP = (X_MAX - X_MIN) / (NUM_KNOT - 1)
INV_STEP = 1.0 / STEP

ROWS = 16384
NW = 32
COLS_PER_W = NUM_ACT // NW
TAB_W = COLS_PER_W * NUM_KNOT
CHUNK_R = 256
N_CHUNKS = ROWS // CHUNK_R
GROUPS_PER_ROW = COLS_PER_W // 16


def _spline_body(x_hbm, coef_hbm, out_hbm, tab_v, xbufs, obufs, sems_in, sems_out):
    cid = lax.axis_index("c")
    sid = lax.axis_index("s")
    wid = sid * 2 + cid
    cb = wid * COLS_PER_W

    pltpu.sync_copy(coef_hbm.at[pl.ds(wid * TAB_W, TAB_W)], tab_v)

    def in_copy(g, b):
        return pltpu.make_async_copy(
            x_hbm.at[pl.ds(g * CHUNK_R, CHUNK_R), pl.ds(cb, COLS_PER_W)],
            xbufs[b],
            sems_in[b],
        )

    def out_copy(g, b):
        return pltpu.make_async_copy(
            obufs[b],
            out_hbm.at[pl.ds(g * CHUNK_R, CHUNK_R), pl.ds(cb, COLS_PER_W)],
            sems_out[b],
        )

    def compute_chunk(b):
        xbuf, obuf = xbufs[b], obufs[b]

        @plsc.parallel_loop(0, CHUNK_R, unroll=4)
        def row_body(r):
            for grp in range(GROUPS_PER_ROW):
                xv = xbuf[r, pl.ds(grp * 16, 16)]
                t = (xv - X_MIN) * INV_STEP
                tcl = jnp.minimum(jnp.maximum(t, 0.0), float(NUM_KNOT - 2))
                ki = tcl.astype(jnp.int32)
                frac = t - ki.astype(jnp.float32)
                idx = (lax.iota(jnp.int32, 16) + grp * 16) * NUM_KNOT + ki
                c0 = plsc.load_gather(tab_v, [idx])
                c1 = plsc.load_gather(tab_v, [idx + 1])
                obuf[r, pl.ds(grp * 16, 16)] = c1 * frac + c0 * (1.0 - frac)

    in_copy(0, 0).start()

    def pair_body(h, carry):
        for b in range(2):
            g = 2 * h + b
            @pl.when(g + 1 < N_CHUNKS)
            def _():
                in_copy(g + 1, 1 - b).start()

            in_copy(g, b).wait()

            @pl.when(g >= 2)
            def _():
                out_copy(jnp.maximum(g - 2, 0), b).wait()

            compute_chunk(b)
            out_copy(g, b).start()
        return carry

    lax.fori_loop(0, N_CHUNKS // 2, pair_body, 0)

    out_copy(N_CHUNKS - 2, 0).wait()
    out_copy(N_CHUNKS - 1, 1).wait()


@jax.jit
def kernel(x, coefficients):
    mesh = plsc.VectorSubcoreMesh(core_axis_name="c", subcore_axis_name="s")
    run = pl.kernel(
        _spline_body,
        out_type=jax.ShapeDtypeStruct((ROWS, NUM_ACT), jnp.float32),
        mesh=mesh,
        scratch_types=[
            pltpu.VMEM((TAB_W,), jnp.float32),
            [pltpu.VMEM((CHUNK_R, COLS_PER_W), jnp.float32) for _ in range(2)],
            [pltpu.VMEM((CHUNK_R, COLS_PER_W), jnp.float32) for _ in range(2)],
            [pltpu.SemaphoreType.DMA for _ in range(2)],
            [pltpu.SemaphoreType.DMA for _ in range(2)],
        ],
        compiler_params=pltpu.CompilerParams(
            use_tc_tiling_on_sc=False, needs_layout_passes=False
        ),
        name="linear_spline_sc",
    )
    return run(x, coefficients.reshape(-1))
